# baseline (speedup 1.0000x reference)
import jax
import jax.numpy as jnp
from jax.experimental import pallas as pl
from jax.experimental.pallas import tpu as pltpu

_B = 20000
_CS = 2000


def _insert_body(state_ref, maskv_ref, maskw_hbm, ns2_ref, ns_ref,
                 out_ref, outm_ref, carry_ref, mchunk_ref, dsem):
    i = pl.program_id(0)
    nb = ns_ref.shape[0]
    b = _B

    @pl.when(i == 0)
    def _():
        carry_ref[0] = 0

    c0 = carry_ref[0]
    m2 = maskv_ref[0]
    e2 = (~m2).astype(jnp.int32)
    zeros = jnp.sum(e2)

    aligned = c0 % 4 == 0
    cond_copy = jnp.logical_or(c0 >= nb, zeros == 0)
    cond_fast = jnp.logical_and(
        jnp.logical_and(zeros == b, c0 + b <= nb), aligned)
    cond_gen = jnp.logical_not(jnp.logical_or(cond_copy, cond_fast))

    @pl.when(cond_copy)
    def _():
        out_ref[...] = state_ref[...]
        outm_ref[...] = maskv_ref[...]

    @pl.when(cond_fast)
    def _():
        out_ref[...] = ns2_ref[pl.ds(c0 // 4, b // 4), :]
        outm_ref[...] = jnp.ones_like(outm_ref)

    @pl.when(cond_gen)
    def _():
        out_ref[...] = state_ref[...]
        lane = jax.lax.broadcasted_iota(jnp.int32, (1, b), 1)
        x = e2
        off = 1
        while off < b:
            x = x + jnp.where(lane >= off, jnp.roll(x, off, axis=1), 0)
            off *= 2
        excl = x - e2
        cnt = c0 + excl
        ins = jnp.logical_and(e2 > 0, cnt < nb)
        outm_ref[...] = jnp.logical_or(m2, ins).reshape(outm_ref.shape)

        fo = jnp.min(jnp.where(m2, lane, b))
        run = jnp.minimum(fo, nb - c0)
        run_vec = jnp.where(aligned, (run >> 2) << 2, 0)
        for k in range(14, 1, -1):
            sz = 1 << k
            if sz > b or sz > nb:
                continue
            done = (run_vec >> (k + 1)) << (k + 1)

            @pl.when(((run_vec >> k) & 1) == 1)
            def _():
                out_ref[pl.ds(done >> 2, sz >> 2), :] = (
                    ns2_ref[pl.ds((c0 + done) >> 2, sz >> 2), :])

        carry_ref[1] = c0 + run_vec
        first_chunk = run_vec // _CS

        def chunk_body(ci, _):
            @pl.when(jnp.logical_and(ci >= first_chunk, carry_ref[1] < nb))
            def _():
                fetch = mchunk_ref.shape[0]
                start = i * b + ci * _CS
                clamp = ((maskw_hbm.shape[0] - fetch) // 128) * 128
                astart = pl.multiple_of(
                    jnp.minimum((start // 128) * 128, clamp), 128)
                delta = start - astart
                cp = pltpu.make_async_copy(
                    maskw_hbm.at[pl.ds(astart, fetch)],
                    mchunk_ref,
                    dsem)
                cp.start()
                cp.wait()

                def row_body(r, _):
                    ra = ci * _CS + r
                    em = jnp.logical_and(mchunk_ref[delta + r] == 0,
                                         ra >= run_vec)
                    c = carry_ref[1]

                    @pl.when(jnp.logical_and(em, c < nb))
                    def _():
                        row = ns_ref[pl.ds(c, 1), :]
                        rq = ra // 4
                        for kk in range(4):
                            @pl.when(ra % 4 == kk)
                            def _():
                                out_ref[pl.ds(rq, 1),
                                        kk * 32:(kk + 1) * 32] = row

                    @pl.when(em)
                    def _():
                        carry_ref[1] = c + 1

                    return 0

                jax.lax.fori_loop(0, _CS, row_body, 0)
            return 0

        jax.lax.fori_loop(0, b // _CS, chunk_body, 0)

    carry_ref[0] = c0 + zeros


def kernel(state, mask, new_states):
    m, d = state.shape
    nb = new_states.shape[0]
    g = m // _B
    state2 = state.reshape(m // 4, 4 * d)
    ns2 = new_states.reshape(nb // 4, 4 * d)
    mask3 = mask.reshape(g, 1, _B)
    mask_i32 = jnp.pad(mask.astype(jnp.int32), (0, (-m) % 128),
                       constant_values=1)

    out_state2, out_mask3 = pl.pallas_call(
        _insert_body,
        grid=(g,),
        in_specs=[
            pl.BlockSpec((_B // 4, 4 * d), lambda i: (i, 0)),
            pl.BlockSpec((1, 1, _B), lambda i: (i, 0, 0)),
            pl.BlockSpec(memory_space=pl.ANY),
            pl.BlockSpec((nb // 4, 4 * d), lambda i: (0, 0)),
            pl.BlockSpec((nb, d), lambda i: (0, 0)),
        ],
        out_specs=[
            pl.BlockSpec((_B // 4, 4 * d), lambda i: (i, 0)),
            pl.BlockSpec((1, 1, _B), lambda i: (i, 0, 0)),
        ],
        out_shape=[
            jax.ShapeDtypeStruct((m // 4, 4 * d), state.dtype),
            jax.ShapeDtypeStruct((g, 1, _B), jnp.bool_),
        ],
        scratch_shapes=[
            pltpu.SMEM((4,), jnp.int32),
            pltpu.SMEM((-(-(_CS + 128) // 128) * 128,), jnp.int32),
            pltpu.SemaphoreType.DMA,
        ],
    )(state2, mask3, mask_i32, ns2, new_states)
    return out_state2.reshape(m, d), out_mask3.reshape(m)


# speedup vs baseline: 1.1512x; 1.1512x over previous
import jax
import jax.numpy as jnp
from jax.experimental import pallas as pl
from jax.experimental.pallas import tpu as pltpu

_B = 20000
_CS = 2000


def _insert_body(state_ref, maskv_ref, maskw_hbm, ns_ref,
                 out_ref, outm_ref, carry_ref, mchunk_ref, dsem):
    i = pl.program_id(0)
    nb = ns_ref.shape[0]
    b = _B

    @pl.when(i == 0)
    def _():
        carry_ref[0] = 0

    c0 = carry_ref[0]
    m2 = maskv_ref[0]
    e2 = (~m2).astype(jnp.int32)
    zeros = jnp.sum(e2)

    cond_copy = jnp.logical_or(c0 >= nb, zeros == 0)
    cond_fast = jnp.logical_and(zeros == b, c0 + b <= nb)
    cond_gen = jnp.logical_not(jnp.logical_or(cond_copy, cond_fast))

    @pl.when(cond_copy)
    def _():
        out_ref[...] = state_ref[...]
        outm_ref[...] = maskv_ref[...]

    @pl.when(cond_fast)
    def _():
        out_ref[...] = ns_ref[pl.ds(c0, b), :]
        outm_ref[...] = jnp.ones_like(outm_ref)

    @pl.when(cond_gen)
    def _():
        out_ref[...] = state_ref[...]
        lane = jax.lax.broadcasted_iota(jnp.int32, (1, b), 1)
        x = e2
        off = 1
        while off < b:
            x = x + jnp.where(lane >= off, jnp.roll(x, off, axis=1), 0)
            off *= 2
        excl = x - e2
        cnt = c0 + excl
        ins = jnp.logical_and(e2 > 0, cnt < nb)
        outm_ref[...] = jnp.logical_or(m2, ins).reshape(outm_ref.shape)

        fo = jnp.min(jnp.where(m2, lane, b))
        run = jnp.minimum(fo, nb - c0)
        for k in range(14, -1, -1):
            sz = 1 << k
            if sz > b or sz > nb:
                continue
            done = (run >> (k + 1)) << (k + 1)

            @pl.when(((run >> k) & 1) == 1)
            def _():
                out_ref[pl.ds(done, sz), :] = ns_ref[pl.ds(c0 + done, sz), :]

        carry_ref[1] = c0 + run
        first_chunk = run // _CS

        def chunk_body(ci, _):
            @pl.when(jnp.logical_and(ci >= first_chunk, carry_ref[1] < nb))
            def _():
                fetch = mchunk_ref.shape[0]
                start = i * b + ci * _CS
                clamp = ((maskw_hbm.shape[0] - fetch) // 128) * 128
                astart = pl.multiple_of(
                    jnp.minimum((start // 128) * 128, clamp), 128)
                delta = start - astart
                cp = pltpu.make_async_copy(
                    maskw_hbm.at[pl.ds(astart, fetch)],
                    mchunk_ref,
                    dsem)
                cp.start()
                cp.wait()

                def row_body(r, _):
                    ra = ci * _CS + r
                    em = jnp.logical_and(mchunk_ref[delta + r] == 0,
                                         ra >= run)
                    c = carry_ref[1]

                    @pl.when(jnp.logical_and(em, c < nb))
                    def _():
                        out_ref[pl.ds(ra, 1), :] = ns_ref[pl.ds(c, 1), :]

                    @pl.when(em)
                    def _():
                        carry_ref[1] = c + 1

                    return 0

                jax.lax.fori_loop(0, _CS, row_body, 0)
            return 0

        jax.lax.fori_loop(0, b // _CS, chunk_body, 0)

    carry_ref[0] = c0 + zeros


def kernel(state, mask, new_states):
    m, d = state.shape
    nb = new_states.shape[0]
    g = m // _B
    mask3 = mask.reshape(g, 1, _B)
    mask_i32 = jnp.pad(mask.astype(jnp.int32), (0, (-m) % 128),
                       constant_values=1)

    out_state, out_mask3 = pl.pallas_call(
        _insert_body,
        grid=(g,),
        in_specs=[
            pl.BlockSpec((_B, d), lambda i: (i, 0)),
            pl.BlockSpec((1, 1, _B), lambda i: (i, 0, 0)),
            pl.BlockSpec(memory_space=pl.ANY),
            pl.BlockSpec((nb, d), lambda i: (0, 0)),
        ],
        out_specs=[
            pl.BlockSpec((_B, d), lambda i: (i, 0)),
            pl.BlockSpec((1, 1, _B), lambda i: (i, 0, 0)),
        ],
        out_shape=[
            jax.ShapeDtypeStruct((m, d), state.dtype),
            jax.ShapeDtypeStruct((g, 1, _B), jnp.bool_),
        ],
        scratch_shapes=[
            pltpu.SMEM((4,), jnp.int32),
            pltpu.SMEM((-(-(_CS + 128) // 128) * 128,), jnp.int32),
            pltpu.SemaphoreType.DMA,
        ],
    )(state, mask3, mask_i32, new_states)
    return out_state, out_mask3.reshape(m)
